# Initial kernel scaffold; baseline (speedup 1.0000x reference)
import jax
import jax.numpy as jnp
from jax.experimental import pallas as pl


def kernel(x, edge_attr, dif_x, dif_edge_attr, params, edge_index, dif_edge_index, label):
    raise NotImplementedError("write your pallas kernel here")


# speedup vs baseline: 24.4262x; 24.4262x over previous
import functools

import numpy as np
import jax
import jax.numpy as jnp
from jax import lax
from jax.experimental import pallas as pl
from jax.experimental.pallas import tpu as pltpu
from jax.experimental.pallas import tpu_sc as plsc

_N = 10000
_E = 320000
_ND = 128
_ED = 16
_HID = 64
_L = 3
_H = 4
_D = 16

_NW = 32
_EW = _E // _NW
_CH = 80
_BN = 400
_NB = _N // _BN

_f32 = jnp.float32
_i32 = jnp.int32

_NUM_MASK = _N // 2
_ZERO_VEC = np.zeros((_N, 1), np.float32)
_THR = 10000
_M_LP = 20480


def _rng_consts():
    key = jax.random.key(42)
    perm = jax.random.permutation(jax.random.fold_in(key, 1), _N)
    mask_nodes = perm[:_NUM_MASK]
    mvec = jnp.zeros((_N,), _f32).at[mask_nodes].set(1.0).reshape(_N, 1)
    neg_src = jax.random.randint(jax.random.fold_in(key, 2), (_THR,), 0, _N, dtype=_i32)
    neg_dst = jax.random.randint(jax.random.fold_in(key, 3), (_THR,), 0, _N, dtype=_i32)
    pos_idx = jax.random.choice(jax.random.fold_in(key, 4), _E, (_THR,), replace=False).astype(_i32)
    pos_pad = jnp.concatenate([pos_idx, jnp.zeros((240,), _i32)])
    mi = jax.random.permutation(jax.random.fold_in(key, 5), _N)[:1024].astype(_i32)
    bi = jax.random.permutation(jax.random.fold_in(key, 6), _N)[:1024].astype(_i32)
    return mvec, neg_src, neg_dst, pos_pad, mi, bi

_S = np.zeros((_HID, _H), np.float32)
for _c in range(_HID):
    _S[_c, _c // _D] = 1.0
_ST = np.ascontiguousarray(_S.T)


_SC_PARAMS = pltpu.CompilerParams(needs_layout_passes=False,
                                  use_tc_tiling_on_sc=False)


def _make_edge_kernel(H, HD):
    ROW = HD + 16
    HPH = HD // H
    NJ = _CH // 16
    mesh = plsc.VectorSubcoreMesh(core_axis_name="c", subcore_axis_name="s", num_cores=2, num_subcores=16)

    @functools.partial(
        pl.kernel,
        out_type=jax.ShapeDtypeStruct((2, _N, ROW), _f32),
        mesh=mesh,
        compiler_params=_SC_PARAMS,
        scratch_types=[
            pltpu.VMEM_SHARED((_N, ROW), _f32),
            pltpu.VMEM((_CH,), _i32),
            pltpu.VMEM((_CH,), _i32),
            pltpu.VMEM((_CH * H,), _f32),
            pltpu.VMEM((_CH, ROW), _f32),
            pltpu.VMEM((_CH, 16), _f32),
            pltpu.VMEM((_CH, ROW), _f32),
            pltpu.VMEM((16, ROW), _f32),
            pltpu.SemaphoreType.DMA,
        ],
    )
    def edge_kernel(htab, ertab, ee, src, dst, out,
                    acc, srcb, dstb, eeb, rowsb, erb, outb, zb, sem):
        cid = lax.axis_index("c")
        sid = lax.axis_index("s")
        wid = cid * 16 + sid
        lanes0 = jnp.arange(16, dtype=_i32)
        zeros16 = jnp.zeros((16,), _f32)

        for c in range(ROW):
            plsc.store_scatter(zb, [lanes0, jnp.full((16,), c, _i32)], zeros16)
        niter = jnp.where(sid == 15, 25, 40)

        def zbody(i, carry):
            r = sid * 640 + i * 16
            pltpu.sync_copy(zb, acc.at[pl.ds(r, 16)])
            return carry
        lax.fori_loop(0, niter, zbody, 0)

        for c in range(H, 16):
            for j in range(NJ):
                plsc.store_scatter(
                    outb, [lanes0 + j * 16, jnp.full((16,), c, _i32)], zeros16)

        plsc.subcore_barrier()

        def ebody(g, carry):
            base = pl.multiple_of(wid * _EW + g * _CH, 8)
            pltpu.sync_copy(src.at[pl.ds(base, _CH)], srcb)
            pltpu.sync_copy(dst.at[pl.ds(base, _CH)], dstb)
            pltpu.sync_copy(ee.at[pl.ds(pl.multiple_of(base * H, 8), _CH * H)], eeb)
            cpa = pltpu.make_async_copy(htab.at[srcb], rowsb, sem)
            cpa.start()
            cpa.wait()
            cpb = pltpu.make_async_copy(ertab.at[dstb], erb, sem)
            cpb.start()
            cpb.wait()
            for j in range(NJ):
                lanes = lanes0 + j * 16
                for h in range(H):
                    el = plsc.load_gather(rowsb, [lanes, jnp.full((16,), HD + h, _i32)])
                    er = plsc.load_gather(erb, [lanes, jnp.full((16,), h, _i32)])
                    ev = plsc.load_gather(eeb, [lanes * H + h])
                    e = el + er + ev
                    e = jnp.maximum(e, 0.2 * e)
                    e = jnp.minimum(e, jnp.float32(60.0))
                    ex = jnp.exp(e)
                    plsc.store_scatter(outb, [lanes, jnp.full((16,), h, _i32)], ex)
                    for t in range(HPH):
                        c = h * HPH + t
                        hv = plsc.load_gather(rowsb, [lanes, jnp.full((16,), c, _i32)])
                        plsc.store_scatter(
                            outb, [lanes, jnp.full((16,), 16 + c, _i32)], hv * ex)
            pltpu.sync_copy(outb, acc.at[dstb], add=True)
            return carry
        lax.fori_loop(0, _EW // _CH, ebody, 0)

        plsc.subcore_barrier()

        def obody(i, carry):
            r = sid * 640 + i * 16
            pltpu.sync_copy(acc.at[pl.ds(r, 16)], zb)
            pltpu.sync_copy(zb, out.at[cid, pl.ds(r, 16)])
            return carry
        lax.fori_loop(0, niter, obody, 0)

    return edge_kernel


_edge_enc = _make_edge_kernel(_H, _HID)
_edge_dec = _make_edge_kernel(1, _ND)


def _make_pair_gather(NT, C, M, CH2, dtype):
    PW = M // _NW
    NCH = PW // CH2
    mesh = plsc.VectorSubcoreMesh(core_axis_name="c", subcore_axis_name="s", num_cores=2, num_subcores=16)

    @functools.partial(
        pl.kernel,
        out_type=(jax.ShapeDtypeStruct((M, C), dtype),
                  jax.ShapeDtypeStruct((M, C), dtype)),
        mesh=mesh,
        compiler_params=_SC_PARAMS,
        scratch_types=[
            pltpu.VMEM((CH2,), _i32),
            pltpu.VMEM((CH2, C), dtype),
            pltpu.SemaphoreType.DMA,
        ],
    )
    def gather_kernel(tabA, idxA, tabB, idxB, outA, outB, idxb, rows, sem):
        wid = lax.axis_index("c") * 16 + lax.axis_index("s")

        def body(g, carry):
            base = pl.multiple_of(wid * PW + g * CH2, 8)
            pltpu.sync_copy(idxA.at[pl.ds(base, CH2)], idxb)
            cp = pltpu.make_async_copy(tabA.at[idxb], rows, sem)
            cp.start()
            cp.wait()
            pltpu.sync_copy(rows, outA.at[pl.ds(base, CH2)])
            pltpu.sync_copy(idxB.at[pl.ds(base, CH2)], idxb)
            cp = pltpu.make_async_copy(tabB.at[idxb], rows, sem)
            cp.start()
            cp.wait()
            pltpu.sync_copy(rows, outB.at[pl.ds(base, CH2)])
            return carry
        lax.fori_loop(0, NCH, body, 0)

    return gather_kernel


def _make_pair_gather1d(M, CH2):
    PW = M // _NW
    NCH = PW // CH2
    NG = CH2 // 16
    mesh = plsc.VectorSubcoreMesh(core_axis_name="c", subcore_axis_name="s", num_cores=2, num_subcores=16)

    @functools.partial(
        pl.kernel,
        out_type=(jax.ShapeDtypeStruct((M,), _i32),
                  jax.ShapeDtypeStruct((M,), _i32)),
        mesh=mesh,
        compiler_params=_SC_PARAMS,
        scratch_types=[
            pltpu.VMEM((CH2,), _i32),
            pltpu.VMEM((CH2,), _i32),
            pltpu.VMEM((CH2, 16), _i32),
            pltpu.VMEM((CH2,), _i32),
            pltpu.SemaphoreType.DMA,
        ],
    )
    def gather1d_kernel(tabA, idxA, tabB, idxB, outA, outB,
                        idxb, rowb, rows, vals, sem):
        wid = lax.axis_index("c") * 16 + lax.axis_index("s")
        lanes0 = jnp.arange(16, dtype=_i32)

        def one(tab, idx, out, base):
            pltpu.sync_copy(idx.at[pl.ds(base, CH2)], idxb)
            for k in range(NG):
                pv = idxb[pl.ds(k * 16, 16)]
                rowb[pl.ds(k * 16, 16)] = pv >> 4
            cp = pltpu.make_async_copy(tab.at[rowb], rows, sem)
            cp.start()
            cp.wait()
            for k in range(NG):
                pv = idxb[pl.ds(k * 16, 16)]
                ev = plsc.load_gather(rows, [lanes0 + k * 16, pv & 15])
                vals[pl.ds(k * 16, 16)] = ev
            pltpu.sync_copy(vals, out.at[pl.ds(base, CH2)])

        def body(g, carry):
            base = pl.multiple_of(wid * PW + g * CH2, 8)
            one(tabA, idxA, outA, base)
            one(tabB, idxB, outB, base)
            return carry
        lax.fori_loop(0, NCH, body, 0)

    return gather1d_kernel


_lp_prologue = _make_pair_gather1d(10240, 64)
_lp_gather = _make_pair_gather(_N, _L * _HID, _M_LP, 128, _f32)
_ct_gather = _make_pair_gather(_N, _L * _HID, 1024, 32, _f32)


def _rows_spec(c):
    return pl.BlockSpec((_BN, c), lambda i: (i, 0))


def _full_spec(shape):
    nd = len(shape)
    return pl.BlockSpec(shape, lambda i: (0,) * nd)


def _dot(a, b):
    return jnp.dot(a, b, preferred_element_type=_f32)


def _ln(v):
    m = jnp.mean(v, axis=-1, keepdims=True)
    var = jnp.mean((v - m) ** 2, axis=-1, keepdims=True)
    return (v - m) / jnp.sqrt(var + 1e-5)


def _node0_body(x_ref, m_ref, mt_ref, W_ref, Wal_ref, War_ref, Wres_ref,
                htab_ref, ertab_ref, res_ref):
    x = x_ref[...]
    m = m_ref[...]
    xm = x * (1.0 - m) + m * mt_ref[...]
    h = _dot(xm, W_ref[...])
    htab_ref[:, 0:_HID] = h
    htab_ref[:, _HID:_HID + _H] = _dot(xm, Wal_ref[...])
    htab_ref[:, _HID + _H:] = jnp.zeros((_BN, 12), _f32)
    ertab_ref[:, 0:_H] = _dot(xm, War_ref[...])
    ertab_ref[:, _H:] = jnp.zeros((_BN, 12), _f32)
    res_ref[...] = _dot(xm, Wres_ref[...])


_node0 = pl.pallas_call(
    _node0_body,
    grid=(_NB,),
    in_specs=[_rows_spec(_ND), _rows_spec(1), _full_spec((1, _ND)),
              _full_spec((_ND, _HID)), _full_spec((_ND, _H)),
              _full_spec((_ND, _H)), _full_spec((_ND, _HID))],
    out_specs=[_rows_spec(_HID + 16), _rows_spec(16), _rows_spec(_HID)],
    out_shape=[jax.ShapeDtypeStruct((_N, _HID + 16), _f32),
               jax.ShapeDtypeStruct((_N, 16), _f32),
               jax.ShapeDtypeStruct((_N, _HID), _f32)],
)


def _node_common(acc_ref, res_ref, b_ref, ST_ref):
    a0 = acc_ref[0]
    a1 = acc_ref[1]
    den = a0[:, 0:_H] + a1[:, 0:_H]
    U = a0[:, 16:16 + _HID] + a1[:, 16:16 + _HID]
    dexp = _dot(den, ST_ref[...])
    o = U / (dexp + 1e-9) + b_ref[...] + res_ref[...]
    return _ln(jnp.maximum(o, 0.0))


def _nodemid_body(acc_ref, res_ref, b_ref, ST_ref, W_ref, Wal_ref, War_ref,
                  y_ref, htab_ref, ertab_ref):
    y = _node_common(acc_ref, res_ref, b_ref, ST_ref)
    y_ref[...] = y
    htab_ref[:, 0:_HID] = _dot(y, W_ref[...])
    htab_ref[:, _HID:_HID + _H] = _dot(y, Wal_ref[...])
    htab_ref[:, _HID + _H:] = jnp.zeros((_BN, 12), _f32)
    ertab_ref[:, 0:_H] = _dot(y, War_ref[...])
    ertab_ref[:, _H:] = jnp.zeros((_BN, 12), _f32)


_nodemid = pl.pallas_call(
    _nodemid_body,
    grid=(_NB,),
    in_specs=[pl.BlockSpec((2, _BN, _HID + 16), lambda i: (0, i, 0)),
              _rows_spec(_HID), _full_spec((1, _HID)), _full_spec((_H, _HID)),
              _full_spec((_HID, _HID)), _full_spec((_HID, _H)),
              _full_spec((_HID, _H))],
    out_specs=[_rows_spec(_HID), _rows_spec(_HID + 16), _rows_spec(16)],
    out_shape=[jax.ShapeDtypeStruct((_N, _HID), _f32),
               jax.ShapeDtypeStruct((_N, _HID + 16), _f32),
               jax.ShapeDtypeStruct((_N, 16), _f32)],
)


def _nodelast_body(acc_ref, res_ref, b_ref, ST_ref, y_ref):
    y_ref[...] = _node_common(acc_ref, res_ref, b_ref, ST_ref)


_nodelast = pl.pallas_call(
    _nodelast_body,
    grid=(_NB,),
    in_specs=[pl.BlockSpec((2, _BN, _HID + 16), lambda i: (0, i, 0)),
              _rows_spec(_HID), _full_spec((1, _HID)), _full_spec((_H, _HID))],
    out_specs=[_rows_spec(_HID)],
    out_shape=[jax.ShapeDtypeStruct((_N, _HID), _f32)],
)


def _decprep_body(y1_ref, y2_ref, y3_ref, We2d_ref, W_ref, Wal_ref, War_ref,
                  Wres_ref, htab_ref, ertab_ref, res_ref):
    We2d = We2d_ref[...]
    rep = (_dot(y1_ref[...], We2d[0:_HID]) +
           _dot(y2_ref[...], We2d[_HID:2 * _HID]) +
           _dot(y3_ref[...], We2d[2 * _HID:]))
    htab_ref[:, 0:_ND] = _dot(rep, W_ref[...])
    htab_ref[:, _ND:_ND + 1] = _dot(rep, Wal_ref[...])
    htab_ref[:, _ND + 1:] = jnp.zeros((_BN, 15), _f32)
    ertab_ref[:, 0:1] = _dot(rep, War_ref[...])
    ertab_ref[:, 1:] = jnp.zeros((_BN, 15), _f32)
    res_ref[...] = _dot(rep, Wres_ref[...])


_decprep = pl.pallas_call(
    _decprep_body,
    grid=(_NB,),
    in_specs=[_rows_spec(_HID), _rows_spec(_HID), _rows_spec(_HID),
              _full_spec((_L * _HID, _HID)), _full_spec((_HID, _ND)),
              _full_spec((_HID, 1)), _full_spec((_HID, 1)),
              _full_spec((_HID, _ND))],
    out_specs=[_rows_spec(_ND + 16), _rows_spec(16), _rows_spec(_ND)],
    out_shape=[jax.ShapeDtypeStruct((_N, _ND + 16), _f32),
               jax.ShapeDtypeStruct((_N, 16), _f32),
               jax.ShapeDtypeStruct((_N, _ND), _f32)],
)


def _decfin_body(acc_ref, res_ref, x_ref, m_ref, b_ref, out_ref):
    a0 = acc_ref[0]
    a1 = acc_ref[1]
    den = a0[:, 0:1] + a1[:, 0:1]
    U = a0[:, 16:] + a1[:, 16:]
    recon = U / (den + 1e-9) + b_ref[...] + res_ref[...]
    x = x_ref[...]
    dot = jnp.sum(recon * x, axis=-1, keepdims=True)
    nr = jnp.sqrt(jnp.sum(recon * recon, axis=-1, keepdims=True))
    nx = jnp.sqrt(jnp.sum(x * x, axis=-1, keepdims=True))
    cos = dot / ((nr + 1e-8) * (nx + 1e-8))
    part = jnp.sum(m_ref[...] * (1.0 - cos) ** 2) * (1.0 / _NUM_MASK)

    @pl.when(pl.program_id(0) == 0)
    def _():
        out_ref[...] = jnp.zeros((1, 1), _f32)
    out_ref[...] += part.reshape(1, 1)


_decfin = pl.pallas_call(
    _decfin_body,
    grid=(_NB,),
    in_specs=[pl.BlockSpec((2, _BN, _ND + 16), lambda i: (0, i, 0)),
              _rows_spec(_ND), _rows_spec(_ND), _rows_spec(1),
              _full_spec((1, _ND))],
    out_specs=pl.BlockSpec((1, 1), lambda i: (0, 0)),
    out_shape=jax.ShapeDtypeStruct((1, 1), _f32),
)


def _sup_body(enc_ref, W1_ref, b1_ref, W2_ref, b2_ref, oh_ref, out_ref):
    hc = jnp.maximum(_dot(enc_ref[...], W1_ref[...]) + b1_ref[...], 0.0)
    logits = _dot(hc, W2_ref[...]) + b2_ref[...]
    mx = jnp.max(logits, axis=-1, keepdims=True)
    lse = mx + jnp.log(jnp.sum(jnp.exp(logits - mx), axis=-1, keepdims=True))
    pick = jnp.sum(logits * oh_ref[...], axis=-1, keepdims=True)
    part = jnp.sum(lse - pick) * (1.0 / _N)

    @pl.when(pl.program_id(0) == 0)
    def _():
        out_ref[...] = jnp.zeros((1, 1), _f32)
    out_ref[...] += part.reshape(1, 1)


_suploss = pl.pallas_call(
    _sup_body,
    grid=(_NB,),
    in_specs=[_rows_spec(_L * _HID), _full_spec((_L * _HID, _HID)),
              _full_spec((1, _HID)), _full_spec((_HID, 2)),
              _full_spec((1, 2)), _full_spec((1, 2))],
    out_specs=pl.BlockSpec((1, 1), lambda i: (0, 0)),
    out_shape=jax.ShapeDtypeStruct((1, 1), _f32),
)


def _lp_body(s_ref, t_ref, W1_ref, b1_ref, W2_ref, b2_ref, out_ref):
    W1 = W1_ref[...]
    z = (_dot(s_ref[...], W1[0:_L * _HID]) +
         _dot(t_ref[...], W1[_L * _HID:]) + b1_ref[...])
    hfc = jnp.maximum(z, 0.2 * z)
    logit = _dot(hfc, W2_ref[...]) + b2_ref[...]
    yp = 1.0 / (1.0 + jnp.exp(-logit))
    yp = jnp.clip(yp, 1e-7, 1.0 - 1e-7)
    i = pl.program_id(0)
    y = jnp.where(i < 25, 1.0, 0.0)
    part = -jnp.sum(y * jnp.log(yp) + (1.0 - y) * jnp.log(1.0 - yp)) * (1.0 / (2 * _THR))

    @pl.when(i == 0)
    def _():
        out_ref[...] = jnp.zeros((1, 1), _f32)
    out_ref[...] += part.reshape(1, 1)


_lp_mlp = pl.pallas_call(
    _lp_body,
    grid=(50,),
    in_specs=[_rows_spec(_L * _HID), _rows_spec(_L * _HID),
              _full_spec((2 * _L * _HID, _HID)), _full_spec((1, _HID)),
              _full_spec((_HID, 1)), _full_spec((1, 1))],
    out_specs=pl.BlockSpec((1, 1), lambda i: (0, 0)),
    out_shape=jax.ShapeDtypeStruct((1, 1), _f32),
)


def _ct_body(me_ref, be_ref, out_ref):
    me = me_ref[...]
    be = be_ref[...]
    dot = jnp.sum(me * be, axis=-1, keepdims=True)
    nm = jnp.sqrt(jnp.sum(me * me, axis=-1, keepdims=True))
    nb = jnp.sqrt(jnp.sum(be * be, axis=-1, keepdims=True))
    cos = dot / (nm * nb + 1e-8)
    part = jnp.sum(jnp.maximum(cos, 0.0)) * (1.0 / 1024)

    @pl.when(pl.program_id(0) == 0)
    def _():
        out_ref[...] = jnp.zeros((1, 1), _f32)
    out_ref[...] += part.reshape(1, 1)


_contrast = pl.pallas_call(
    _ct_body,
    grid=(2,),
    in_specs=[pl.BlockSpec((512, _L * _HID), lambda i: (i, 0)),
              pl.BlockSpec((512, _L * _HID), lambda i: (i, 0))],
    out_specs=pl.BlockSpec((1, 1), lambda i: (0, 0)),
    out_shape=jax.ShapeDtypeStruct((1, 1), _f32),
)


def _ee_body(ea_ref, W0_ref, W1_ref, W2_ref, Wd_ref, o0_ref, o1_ref, o2_ref, od_ref):
    a = ea_ref[...]
    o0_ref[...] = _dot(a, W0_ref[...])
    o1_ref[...] = _dot(a, W1_ref[...])
    o2_ref[...] = _dot(a, W2_ref[...])
    od_ref[...] = _dot(a, Wd_ref[...])


_EB = 2000
_ee_call = pl.pallas_call(
    _ee_body,
    grid=(_E // 8 // _EB,),
    in_specs=[pl.BlockSpec((_EB, 128), lambda i: (i, 0)),
              _full_spec((128, 8 * _H)), _full_spec((128, 8 * _H)),
              _full_spec((128, 8 * _H)), _full_spec((128, 8))],
    out_specs=[pl.BlockSpec((_EB, 8 * _H), lambda i: (i, 0)),
               pl.BlockSpec((_EB, 8 * _H), lambda i: (i, 0)),
               pl.BlockSpec((_EB, 8 * _H), lambda i: (i, 0)),
               pl.BlockSpec((_EB, 8), lambda i: (i, 0))],
    out_shape=[jax.ShapeDtypeStruct((_E // 8, 8 * _H), _f32),
               jax.ShapeDtypeStruct((_E // 8, 8 * _H), _f32),
               jax.ShapeDtypeStruct((_E // 8, 8 * _H), _f32),
               jax.ShapeDtypeStruct((_E // 8, 8), _f32)],
)


def _fold_enc(p, l):
    S = jnp.asarray(_S)
    W = p[f'enc{l}_W']
    Wal = _dot(W, p[f'enc{l}_al'].reshape(-1)[:, None] * S)
    War = _dot(W, p[f'enc{l}_ar'].reshape(-1)[:, None] * S)
    Me = _dot(p[f'enc{l}_We'], p[f'enc{l}_ae'].reshape(-1)[:, None] * S)
    Wk = jnp.kron(jnp.eye(8, dtype=_f32), Me)
    return W, Wal, War, Wk


def _encode(x, mvec, src, dst, eefs, p, folds):
    (W0, Wal0, War0, _), (W1, Wal1, War1, _), (W2, Wal2, War2, _) = folds
    ST = jnp.asarray(_ST)
    htab0, ertab0, res0 = _node0(x, mvec, p['mask_token'], W0, Wal0, War0,
                                 p['enc0_Wres'])
    acc0 = _edge_enc(htab0, ertab0, eefs[0], src, dst)
    y1, htab1, ertab1 = _nodemid(acc0, res0, p['enc0_b'].reshape(1, -1), ST,
                                 W1, Wal1, War1)
    acc1 = _edge_enc(htab1, ertab1, eefs[1], src, dst)
    y2, htab2, ertab2 = _nodemid(acc1, y1, p['enc1_b'].reshape(1, -1), ST,
                                 W2, Wal2, War2)
    acc2 = _edge_enc(htab2, ertab2, eefs[2], src, dst)
    (y3,) = _nodelast(acc2, y2, p['enc2_b'].reshape(1, -1), ST)
    return y1, y2, y3


def kernel(x, edge_attr, dif_x, dif_edge_attr, params, edge_index,
           dif_edge_index, label):
    p = params
    src, dst = edge_index[0], edge_index[1]
    dsrc, ddst = dif_edge_index[0], dif_edge_index[1]
    mvec, neg_src, neg_dst, pos_pad, mi, bi = _rng_consts()

    folds = [_fold_enc(p, l) for l in range(_L)]
    Wal_d = _dot(p['dec_W'], p['dec_al'].reshape(-1)[:, None])
    War_d = _dot(p['dec_W'], p['dec_ar'].reshape(-1)[:, None])
    Me_d = _dot(p['dec_We'], p['dec_ae'].reshape(-1)[:, None])
    Wkd = jnp.kron(jnp.eye(8, dtype=_f32), Me_d)

    ee0, ee1, ee2, eed = _ee_call(edge_attr.reshape(_E // 8, 128),
                                  folds[0][3], folds[1][3], folds[2][3], Wkd)
    de0, de1, de2, _ = _ee_call(dif_edge_attr.reshape(_E // 8, 128),
                                folds[0][3], folds[1][3], folds[2][3], Wkd)
    eefs = [ee0.reshape(-1), ee1.reshape(-1), ee2.reshape(-1)]
    deefs = [de0.reshape(-1), de1.reshape(-1), de2.reshape(-1)]

    y1, y2, y3 = _encode(x, mvec, src, dst, eefs, p, folds)

    htabd, ertabd, resd = _decprep(y1, y2, y3, p['W_e2d'], p['dec_W'],
                                   Wal_d, War_d, p['dec_Wres'])
    accd = _edge_dec(htabd, ertabd, eed.reshape(-1), src, dst)
    recon_rec = _decfin(accd, resd, x, mvec, p['dec_b'].reshape(1, -1))

    z1, z2, z3 = _encode(dif_x, jnp.asarray(_ZERO_VEC), dsrc, ddst, deefs,
                         p, folds)

    enc_rep = jnp.concatenate([y1, y2, y3], axis=1)
    denc = jnp.concatenate([z1, z2, z3], axis=1)

    srcsel, dstsel = _lp_prologue(src.reshape(_E // 16, 16), pos_pad,
                                  dst.reshape(_E // 16, 16), pos_pad)
    pad480 = jnp.zeros((480,), _i32)
    sidx = jnp.concatenate([srcsel[:_THR], neg_src, pad480])
    tidx = jnp.concatenate([dstsel[:_THR], neg_dst, pad480])
    s_rows, t_rows = _lp_gather(enc_rep, sidx, enc_rep, tidx)
    recon_loss = _lp_mlp(s_rows, t_rows, p['fc1_W'],
                         p['fc1_b'].reshape(1, -1), p['fc2_W'],
                         p['fc2_b'].reshape(1, -1))

    onehot = (jnp.arange(2) == label).astype(_f32).reshape(1, 2)
    sup = _suploss(enc_rep, p['cls1_W'], p['cls1_b'].reshape(1, -1),
                   p['cls2_W'], p['cls2_b'].reshape(1, -1), onehot)

    me, be = _ct_gather(denc, mi, enc_rep, bi)
    contrast = _contrast(me, be)

    return (recon_rec[0, 0] + recon_loss[0, 0] + sup[0, 0] + contrast[0, 0])
